# baseline (speedup 1.0000x reference)
import jax
import jax.numpy as jnp
import numpy as np
from jax.experimental import pallas as pl
from jax.experimental.pallas import tpu as pltpu

_K = 8192
_D = 256
_LOG_K = float(np.log(_K))
_R = 256
_C = 2048
_NCH = _K // _C

_HIGHEST = jax.lax.Precision.HIGHEST
_DEFAULT = jax.lax.Precision.DEFAULT
_NT_DIMS = (((1,), (1,)), ((), ()))


def _vq_body(z_ref, cb_ref, g_ref, zq_ref, hard_ref, kl_ref, cm_ref,
             cc_ref, cb16_ref, dist_ref, e16_ref):
    i = pl.program_id(0)

    @pl.when(i == 0)
    def _init():
        cb = cb_ref[...]
        cb16_ref[...] = cb.astype(jnp.bfloat16)
        kl_ref[...] = jnp.zeros_like(kl_ref)
        cm_ref[...] = jnp.zeros_like(cm_ref)
        ones = jnp.ones((1, _D), jnp.float32)
        cc_ref[...] = jax.lax.dot_general(
            ones, cb * cb, _NT_DIMS, precision=_HIGHEST,
            preferred_element_type=jnp.float32)

    z = z_ref[...]
    z16 = z.astype(jnp.bfloat16)
    zz = jnp.sum(z * z, axis=1, keepdims=True)

    w_max = jnp.full((_R, 1), -jnp.inf, jnp.float32)
    d_min = jnp.full((_R, 1), jnp.inf, jnp.float32)
    for c in range(_NCH):
        ds = pl.ds(c * _C, _C)
        cross = jax.lax.dot_general(
            z16, cb16_ref[ds, :], _NT_DIMS, precision=_DEFAULT,
            preferred_element_type=jnp.float32)
        dist = (cc_ref[:, ds] + zz) - 2.0 * cross
        dist_ref[:, ds] = dist
        w = g_ref[:, ds] - dist
        w_max = jnp.maximum(w_max, jnp.max(w, axis=1, keepdims=True))
        d_min = jnp.minimum(d_min, jnp.min(dist, axis=1, keepdims=True))

    s = jnp.zeros((_R, 1), jnp.float32)
    s2 = jnp.zeros((_R, 1), jnp.float32)
    cd = jnp.zeros((_R, 1), jnp.float32)
    idx = jnp.full((_R, 1), _K, jnp.int32)
    for c in range(_NCH):
        ds = pl.ds(c * _C, _C)
        dist = dist_ref[:, ds]
        w = g_ref[:, ds] - dist
        t = w - w_max
        e = jnp.exp(t + t)
        s = s + jnp.sum(e, axis=1, keepdims=True)
        e16_ref[:, ds] = e.astype(jnp.bfloat16)
        e2 = jnp.exp(d_min - dist)
        s2 = s2 + jnp.sum(e2, axis=1, keepdims=True)
        cd = cd + jnp.sum(e2 * dist, axis=1, keepdims=True)
        lane = jax.lax.broadcasted_iota(jnp.int32, (_R, _C), 1) + (c * _C)
        hit = jnp.where(w == w_max, lane, _K)
        idx = jnp.minimum(idx, jnp.min(hit, axis=1, keepdims=True))

    zq = jax.lax.dot_general(
        e16_ref[...], cb16_ref[...], (((1,), (0,)), ((), ())),
        precision=_DEFAULT, preferred_element_type=jnp.float32)
    hard_ref[...] = idx
    zq_ref[...] = zq * (1.0 / s)

    inv_s2 = 1.0 / s2
    row_cm = cd * inv_s2
    row_kl = (d_min + (_LOG_K - jnp.log(s2))) - row_cm
    kl_ref[...] += jnp.sum(row_kl, keepdims=True)
    cm_ref[...] += jnp.sum(row_cm, keepdims=True)


def kernel(z_e, codebook, gumbel):
    bs, feat, w, h = z_e.shape
    n = bs * w * h
    z = jnp.transpose(z_e, (0, 2, 3, 1)).reshape(n, feat)
    grid = (n // _R,)
    zq, hard, kl, cm = pl.pallas_call(
        _vq_body,
        grid=grid,
        in_specs=[
            pl.BlockSpec((_R, _D), lambda i: (i, 0)),
            pl.BlockSpec((_K, _D), lambda i: (0, 0)),
            pl.BlockSpec((_R, _K), lambda i: (i, 0)),
        ],
        out_specs=[
            pl.BlockSpec((_R, _D), lambda i: (i, 0)),
            pl.BlockSpec((_R, 1), lambda i: (i, 0)),
            pl.BlockSpec((1, 1), lambda i: (0, 0)),
            pl.BlockSpec((1, 1), lambda i: (0, 0)),
        ],
        out_shape=[
            jax.ShapeDtypeStruct((n, _D), jnp.float32),
            jax.ShapeDtypeStruct((n, 1), jnp.int32),
            jax.ShapeDtypeStruct((1, 1), jnp.float32),
            jax.ShapeDtypeStruct((1, 1), jnp.float32),
        ],
        scratch_shapes=[
            pltpu.VMEM((1, _K), jnp.float32),
            pltpu.VMEM((_K, _D), jnp.bfloat16),
            pltpu.VMEM((_R, _K), jnp.float32),
            pltpu.VMEM((_R, _K), jnp.bfloat16),
        ],
    )(z, codebook, gumbel)
    z_q = jnp.transpose(zq.reshape(bs, w, h, feat), (0, 3, 1, 2))
    hard_indices = hard.reshape(bs, w, h)
    inv_bs = np.float32(1.0 / bs)
    return (z_q, hard_indices, kl[0, 0] * inv_bs, cm[0, 0] * inv_bs)


# speedup vs baseline: 1.0472x; 1.0472x over previous
import jax
import jax.numpy as jnp
import numpy as np
from jax.experimental import pallas as pl
from jax.experimental.pallas import tpu as pltpu

_K = 8192
_D = 256
_LOG_K = float(np.log(_K))
_R = 256
_C = 1024
_NCH = _K // _C

_HIGHEST = jax.lax.Precision.HIGHEST
_DEFAULT = jax.lax.Precision.DEFAULT
_NT_DIMS = (((1,), (1,)), ((), ()))


def _vq_body(z_ref, cb_ref, g_ref, zq_ref, hard_ref, kl_ref, cm_ref,
             cc_ref, cb16_ref, dist_ref):
    i = pl.program_id(0)

    @pl.when(i == 0)
    def _init():
        cb = cb_ref[...]
        cb16_ref[...] = cb.astype(jnp.bfloat16)
        kl_ref[...] = jnp.zeros_like(kl_ref)
        cm_ref[...] = jnp.zeros_like(cm_ref)
        ones = jnp.ones((1, _D), jnp.float32)
        cc_ref[...] = jax.lax.dot_general(
            ones, cb * cb, _NT_DIMS, precision=_HIGHEST,
            preferred_element_type=jnp.float32)

    z = z_ref[...]
    z16 = z.astype(jnp.bfloat16)
    zz = jnp.sum(z * z, axis=1, keepdims=True)

    w_max = jnp.full((_R, 1), -jnp.inf, jnp.float32)
    d_min = jnp.full((_R, 1), jnp.inf, jnp.float32)
    for c in range(_NCH):
        ds = pl.ds(c * _C, _C)
        cross = jax.lax.dot_general(
            z16, cb16_ref[ds, :], _NT_DIMS, precision=_DEFAULT,
            preferred_element_type=jnp.float32)
        dist = (cc_ref[:, ds] + zz) - 2.0 * cross
        dist_ref[:, ds] = dist
        w = g_ref[:, ds] - dist
        w_max = jnp.maximum(w_max, jnp.max(w, axis=1, keepdims=True))
        d_min = jnp.minimum(d_min, jnp.min(dist, axis=1, keepdims=True))

    s = jnp.zeros((_R, 1), jnp.float32)
    s2 = jnp.zeros((_R, 1), jnp.float32)
    cd = jnp.zeros((_R, 1), jnp.float32)
    idx = jnp.full((_R, 1), _K, jnp.int32)
    zq = jnp.zeros((_R, _D), jnp.float32)
    for c in range(_NCH):
        ds = pl.ds(c * _C, _C)
        dist = dist_ref[:, ds]
        w = g_ref[:, ds] - dist
        t = w - w_max
        e = jnp.exp(t + t)
        s = s + jnp.sum(e, axis=1, keepdims=True)
        e2 = jnp.exp(d_min - dist)
        s2 = s2 + jnp.sum(e2, axis=1, keepdims=True)
        cd = cd + jnp.sum(e2 * dist, axis=1, keepdims=True)
        lane = jax.lax.broadcasted_iota(jnp.int32, (_R, _C), 1) + (c * _C)
        hit = jnp.where(w == w_max, lane, _K)
        idx = jnp.minimum(idx, jnp.min(hit, axis=1, keepdims=True))
        zq = zq + jax.lax.dot_general(
            e.astype(jnp.bfloat16), cb16_ref[ds, :], (((1,), (0,)), ((), ())),
            precision=_DEFAULT, preferred_element_type=jnp.float32)

    hard_ref[...] = idx
    zq_ref[...] = zq * (1.0 / s)

    inv_s2 = 1.0 / s2
    row_cm = cd * inv_s2
    row_kl = (d_min + (_LOG_K - jnp.log(s2))) - row_cm
    kl_ref[...] += jnp.sum(row_kl, keepdims=True)
    cm_ref[...] += jnp.sum(row_cm, keepdims=True)


def kernel(z_e, codebook, gumbel):
    bs, feat, w, h = z_e.shape
    n = bs * w * h
    z = jnp.transpose(z_e, (0, 2, 3, 1)).reshape(n, feat)
    grid = (n // _R,)
    zq, hard, kl, cm = pl.pallas_call(
        _vq_body,
        grid=grid,
        in_specs=[
            pl.BlockSpec((_R, _D), lambda i: (i, 0)),
            pl.BlockSpec((_K, _D), lambda i: (0, 0)),
            pl.BlockSpec((_R, _K), lambda i: (i, 0)),
        ],
        out_specs=[
            pl.BlockSpec((_R, _D), lambda i: (i, 0)),
            pl.BlockSpec((_R, 1), lambda i: (i, 0)),
            pl.BlockSpec((1, 1), lambda i: (0, 0)),
            pl.BlockSpec((1, 1), lambda i: (0, 0)),
        ],
        out_shape=[
            jax.ShapeDtypeStruct((n, _D), jnp.float32),
            jax.ShapeDtypeStruct((n, 1), jnp.int32),
            jax.ShapeDtypeStruct((1, 1), jnp.float32),
            jax.ShapeDtypeStruct((1, 1), jnp.float32),
        ],
        scratch_shapes=[
            pltpu.VMEM((1, _K), jnp.float32),
            pltpu.VMEM((_K, _D), jnp.bfloat16),
            pltpu.VMEM((_R, _K), jnp.float32),
        ],
    )(z, codebook, gumbel)
    z_q = jnp.transpose(zq.reshape(bs, w, h, feat), (0, 3, 1, 2))
    hard_indices = hard.reshape(bs, w, h)
    inv_bs = np.float32(1.0 / bs)
    return (z_q, hard_indices, kl[0, 0] * inv_bs, cm[0, 0] * inv_bs)


# speedup vs baseline: 1.0531x; 1.0056x over previous
import jax
import jax.numpy as jnp
import numpy as np
from jax.experimental import pallas as pl
from jax.experimental.pallas import tpu as pltpu

_K = 8192
_D = 256
_LOG_K = float(np.log(_K))
_R = 256
_C = 1024
_NCH = _K // _C

_HIGHEST = jax.lax.Precision.HIGHEST
_DEFAULT = jax.lax.Precision.DEFAULT
_NT_DIMS = (((1,), (1,)), ((), ()))


def _vq_body(z_ref, cb_ref, g_ref, zq_ref, hard_ref, kl_ref, cm_ref,
             cc_ref, cb16_ref, dist_ref):
    i = pl.program_id(0)

    @pl.when(i == 0)
    def _init():
        cb = cb_ref[...]
        cb16_ref[...] = cb.astype(jnp.bfloat16)
        kl_ref[...] = jnp.zeros_like(kl_ref)
        cm_ref[...] = jnp.zeros_like(cm_ref)
        ones = jnp.ones((1, _D), jnp.float32)
        cc_ref[...] = jax.lax.dot_general(
            ones, cb * cb, _NT_DIMS, precision=_HIGHEST,
            preferred_element_type=jnp.float32)

    z = z_ref[...]
    z16 = z.astype(jnp.bfloat16)
    zz = jnp.sum(z * z, axis=1, keepdims=True)

    w_max = jnp.full((_R, 1), -jnp.inf, jnp.float32)
    d_min = jnp.full((_R, 1), jnp.inf, jnp.float32)
    for c in range(_NCH):
        ds = pl.ds(c * _C, _C)
        cross = jax.lax.dot_general(
            z16, cb16_ref[ds, :], _NT_DIMS, precision=_DEFAULT,
            preferred_element_type=jnp.float32)
        dist = (cc_ref[:, ds] + zz) - 2.0 * cross
        dist_ref[:, ds] = dist
        w = g_ref[:, ds] - dist
        w_max = jnp.maximum(w_max, jnp.max(w, axis=1, keepdims=True))
        d_min = jnp.minimum(d_min, jnp.min(dist, axis=1, keepdims=True))

    s = jnp.zeros((_R, 1), jnp.float32)
    s2 = jnp.zeros((_R, 1), jnp.float32)
    cd = jnp.zeros((_R, 1), jnp.float32)
    idx = jnp.full((_R, 1), _K, jnp.int32)
    zq = jnp.zeros((_R, _D), jnp.float32)
    for c in range(_NCH):
        ds = pl.ds(c * _C, _C)
        dist = dist_ref[:, ds]
        w = g_ref[:, ds] - dist
        t = w - w_max
        e = jnp.exp(t + t)
        s = s + jnp.sum(e, axis=1, keepdims=True)
        e2 = jnp.exp(d_min - dist)
        s2 = s2 + jnp.sum(e2, axis=1, keepdims=True)
        cd = cd + jnp.sum(e2 * dist, axis=1, keepdims=True)
        lane = jax.lax.broadcasted_iota(jnp.int32, (_R, _C), 1)
        hit = jnp.where(w == w_max, lane, _K)
        idx = jnp.minimum(idx, jnp.min(hit, axis=1, keepdims=True) + (c * _C))
        zq = zq + jax.lax.dot_general(
            e.astype(jnp.bfloat16), cb16_ref[ds, :], (((1,), (0,)), ((), ())),
            precision=_DEFAULT, preferred_element_type=jnp.float32)

    hard_ref[...] = idx
    zq_ref[...] = zq * (1.0 / s)

    inv_s2 = 1.0 / s2
    row_cm = cd * inv_s2
    row_kl = (d_min + (_LOG_K - jnp.log(s2))) - row_cm
    kl_ref[...] += jnp.sum(row_kl, keepdims=True)
    cm_ref[...] += jnp.sum(row_cm, keepdims=True)


def kernel(z_e, codebook, gumbel):
    bs, feat, w, h = z_e.shape
    n = bs * w * h
    z = jnp.transpose(z_e, (0, 2, 3, 1)).reshape(n, feat)
    grid = (n // _R,)
    zq, hard, kl, cm = pl.pallas_call(
        _vq_body,
        grid=grid,
        in_specs=[
            pl.BlockSpec((_R, _D), lambda i: (i, 0)),
            pl.BlockSpec((_K, _D), lambda i: (0, 0)),
            pl.BlockSpec((_R, _K), lambda i: (i, 0)),
        ],
        out_specs=[
            pl.BlockSpec((_R, _D), lambda i: (i, 0)),
            pl.BlockSpec((_R, 1), lambda i: (i, 0)),
            pl.BlockSpec((1, 1), lambda i: (0, 0)),
            pl.BlockSpec((1, 1), lambda i: (0, 0)),
        ],
        out_shape=[
            jax.ShapeDtypeStruct((n, _D), jnp.float32),
            jax.ShapeDtypeStruct((n, 1), jnp.int32),
            jax.ShapeDtypeStruct((1, 1), jnp.float32),
            jax.ShapeDtypeStruct((1, 1), jnp.float32),
        ],
        scratch_shapes=[
            pltpu.VMEM((1, _K), jnp.float32),
            pltpu.VMEM((_K, _D), jnp.bfloat16),
            pltpu.VMEM((_R, _K), jnp.float32),
        ],
    )(z, codebook, gumbel)
    z_q = jnp.transpose(zq.reshape(bs, w, h, feat), (0, 3, 1, 2))
    hard_indices = hard.reshape(bs, w, h)
    inv_bs = np.float32(1.0 / bs)
    return (z_q, hard_indices, kl[0, 0] * inv_bs, cm[0, 0] * inv_bs)
